# baseline (speedup 1.0000x reference)
import functools

import jax
import jax.numpy as jnp
from jax import lax
from jax.experimental import pallas as pl
from jax.experimental.pallas import tpu as pltpu
from jax.experimental.pallas import tpu_sc as plsc

BATCH = 16
N_PRIORS = 8732
N_CLASSES = 81
N_OBJ = 16
THRESHOLD = 0.5
ALPHA = 25.0
EPS = 1e-7

BLK = 2048
NBLK = 5
PADP = BLK * NBLK
PADSC = 8736
NCHUNK = PADSC // 16
NPADCHUNK = (PADP - PADSC) // 16


def _sc_assign_body(boxes_hbm, labels_hbm, priors_hbm, lab_out, tl_out,
                    pri_v, box_v, labl_v, lab_v, tl_v):
    wid = lax.axis_index("s") * 2 + lax.axis_index("c")

    @pl.when(wid < BATCH)
    def _():
        b = wid
        pltpu.sync_copy(boxes_hbm.at[b], box_v)
        pltpu.sync_copy(labels_hbm.at[b], labl_v)
        pltpu.sync_copy(priors_hbm, pri_v)
        lanes = lax.iota(jnp.int32, 16)
        zzf = jnp.zeros((16,), jnp.float32)

        def splat(k, i):
            return zzf + jnp.sum(jnp.where(lanes == i, box_v[k, :], 0.0))

        bx0 = [splat(0, i) for i in range(N_OBJ)]
        by0 = [splat(1, i) for i in range(N_OBJ)]
        bx1 = [splat(2, i) for i in range(N_OBJ)]
        by1 = [splat(3, i) for i in range(N_OBJ)]
        barea = [(bx1[i] - bx0[i]) * (by1[i] - by0[i]) for i in range(N_OBJ)]

        def body(j, carry):
            rn, rd, ri = carry[:N_OBJ], carry[N_OBJ:2 * N_OBJ], carry[2 * N_OBJ:]
            base = j * 16
            pcx = pri_v[0, pl.ds(base, 16)]
            pcy = pri_v[1, pl.ds(base, 16)]
            pw = pri_v[2, pl.ds(base, 16)]
            ph = pri_v[3, pl.ds(base, 16)]
            px0 = pcx - pw * 0.5
            py0 = pcy - ph * 0.5
            px1 = pcx + pw * 0.5
            py1 = pcy + ph * 0.5
            parea = (px1 - px0) * (py1 - py0)
            idxv = base + lanes
            nf = jnp.full((16,), -1.0, jnp.float32)
            df = jnp.ones((16,), jnp.float32)
            of = jnp.zeros((16,), jnp.int32)
            new_rn, new_rd, new_ri = [], [], []
            for i in range(N_OBJ):
                iw = jnp.maximum(jnp.minimum(bx1[i], px1)
                                 - jnp.maximum(bx0[i], px0), 0.0)
                ih = jnp.maximum(jnp.minimum(by1[i], py1)
                                 - jnp.maximum(by0[i], py0), 0.0)
                n = iw * ih
                dd = barea[i] + parea - n + EPS
                better = n * df > nf * dd
                of = jnp.where(better, i, of)
                nf = jnp.where(better, n, nf)
                df = jnp.where(better, dd, df)
                upd = n * rd[i] > rn[i] * dd
                new_rn.append(jnp.where(upd, n, rn[i]))
                new_rd.append(jnp.where(upd, dd, rd[i]))
                new_ri.append(jnp.where(upd, idxv, ri[i]))
            labv = plsc.load_gather(labl_v, [jnp.zeros((16,), jnp.int32), of])
            labv = jnp.where(nf < THRESHOLD * df, -1, labv)
            labv = jnp.where(nf < (THRESHOLD - 0.1) * df, 0, labv)
            labv = jnp.where(idxv < N_PRIORS, labv, -1)
            lab_v[0, pl.ds(base, 16)] = labv
            for k in range(4):
                tl_v[k, pl.ds(base, 16)] = plsc.load_gather(
                    box_v, [jnp.full((16,), k, jnp.int32), of])
            return tuple(new_rn) + tuple(new_rd) + tuple(new_ri)

        init = ((jnp.full((16,), -1.0, jnp.float32),) * N_OBJ
                + (jnp.ones((16,), jnp.float32),) * N_OBJ
                + (jnp.zeros((16,), jnp.int32),) * N_OBJ)
        carry = lax.fori_loop(0, NCHUNK, body, init)
        rn, rd, ri = carry[:N_OBJ], carry[N_OBJ:2 * N_OBJ], carry[2 * N_OBJ:]

        def padbody(j, acc):
            base = PADSC + j * 16
            lab_v[0, pl.ds(base, 16)] = jnp.full((16,), -1, jnp.int32)
            for k in range(4):
                tl_v[k, pl.ds(base, 16)] = jnp.zeros((16,), jnp.float32)
            return acc

        lax.fori_loop(0, NPADCHUNK, padbody, 0)

        mask0 = lanes == 0
        z16 = jnp.zeros((16,), jnp.int32)
        for i in range(N_OBJ):
            bn = jnp.full((16,), -1.0, jnp.float32)
            bd = jnp.ones((16,), jnp.float32)
            bi = jnp.full((16,), PADP, jnp.int32)
            for l in range(16):
                lsel = lanes == l
                nl = zzf + jnp.sum(jnp.where(lsel, rn[i], 0.0))
                dl = zzf + jnp.sum(jnp.where(lsel, rd[i], 0.0))
                il = z16 + jnp.sum(jnp.where(lsel, ri[i], 0))
                cl = nl * bd
                cr = bn * dl
                better = jnp.logical_or(cl > cr,
                                        jnp.logical_and(cl == cr, il < bi))
                bn = jnp.where(better, nl, bn)
                bd = jnp.where(better, dl, bd)
                bi = jnp.where(better, il, bi)
            pfi = jnp.min(bi)
            pfv = jnp.full((16,), 0, jnp.int32) + pfi
            li = z16 + jnp.sum(jnp.where(lanes == i, labl_v[0, :], 0))
            plsc.store_scatter(lab_v, [jnp.zeros((16,), jnp.int32), pfv], li, mask=mask0)
            for k in range(4):
                plsc.store_scatter(tl_v, [jnp.full((16,), k, jnp.int32), pfv],
                                   splat(k, i), mask=mask0)

        pltpu.sync_copy(lab_v, lab_out.at[b])
        pltpu.sync_copy(tl_v, tl_out.at[b])


def _sc_assign(boxes_sc, labels, priors_sc):
    fn = pl.kernel(
        _sc_assign_body,
        mesh=plsc.VectorSubcoreMesh(core_axis_name="c", subcore_axis_name="s"),
        out_type=[jax.ShapeDtypeStruct((BATCH, 1, PADP), jnp.int32),
                  jax.ShapeDtypeStruct((BATCH, 4, PADP), jnp.float32)],
        scratch_types=[
            pltpu.VMEM((4, PADSC), jnp.float32),
            pltpu.VMEM((4, N_OBJ), jnp.float32),
            pltpu.VMEM((1, N_OBJ), jnp.int32),
            pltpu.VMEM((1, PADP), jnp.int32),
            pltpu.VMEM((4, PADP), jnp.float32),
        ],
        compiler_params=pltpu.CompilerParams(needs_layout_passes=False),
    )
    return fn(boxes_sc, labels, priors_sc)


def _mbox_kernel(locs_ref, scores_ref, lab_in, tl_in, priors_ref,
                 out_ref, acc_ref):
    b = pl.program_id(0)
    s = pl.program_id(1)

    @pl.when(jnp.logical_and(b == 0, s == 0))
    def _init():
        acc_ref[...] = jnp.zeros_like(acc_ref)

    g = locs_ref[0, :, :]
    pr = priors_ref[:, pl.ds(s * BLK, BLK)]
    pcx, pcy, pw, ph = pr[0:1], pr[1:2], pr[2:3], pr[3:4]
    cx = g[0:1] * pw / 10.0 + pcx
    cy = g[1:2] * ph / 10.0 + pcy
    w = jnp.exp(g[2:3] / 5.0) * pw
    h = jnp.exp(g[3:4] / 5.0) * ph
    dx0 = cx - w / 2.0
    dy0 = cy - h / 2.0
    dx1 = cx + w / 2.0
    dy1 = cy + h / 2.0

    t = tl_in[0, :, pl.ds(s * BLK, BLK)]
    tx0, ty0, tx1, ty1 = t[0:1], t[1:2], t[2:3], t[3:4]
    inter = (jnp.clip(jnp.minimum(dx1, tx1) - jnp.maximum(dx0, tx0), 0.0, None)
             * jnp.clip(jnp.minimum(dy1, ty1) - jnp.maximum(dy0, ty0), 0.0, None))
    ap = (dx1 - dx0) * (dy1 - dy0)
    at_ = (tx1 - tx0) * (ty1 - ty0)
    iou = inter / (ap + at_ - inter + EPS)
    rho2 = (((dx0 + dx1) - (tx0 + tx1)) / 2.0) ** 2 + (((dy0 + dy1) - (ty0 + ty1)) / 2.0) ** 2
    ex = jnp.maximum(dx1, tx1) - jnp.minimum(dx0, tx0)
    ey = jnp.maximum(dy1, ty1) - jnp.minimum(dy0, ty0)
    c2 = ex * ex + ey * ey + EPS
    diou = 1.0 - (iou - rho2 / c2)

    lab_row = lab_in[0, 0:1, pl.ds(s * BLK, BLK)]
    posr = lab_row > 0
    sd = jnp.sum(jnp.where(posr, diou, 0.0), axis=1, keepdims=True)
    npos = jnp.sum(posr.astype(jnp.float32), axis=1, keepdims=True)

    St = jnp.transpose(scores_ref[0, :, :], (1, 0))
    tgt = jnp.clip(lab_row, 0, N_CLASSES - 1)
    cid = jax.lax.broadcasted_iota(jnp.int32, (N_CLASSES, BLK), 0)
    ones_c = jnp.ones((1, N_CLASSES), jnp.float32)
    se = jax.lax.dot_general(ones_c, jnp.exp(St), (((1,), (0,)), ((), ())),
                             preferred_element_type=jnp.float32)
    s_tgt = jax.lax.dot_general(ones_c, jnp.where(cid == tgt, St, 0.0),
                                (((1,), (0,)), ((), ())),
                                preferred_element_type=jnp.float32)
    logpt = s_tgt - jnp.log(se)
    pt = jnp.exp(logpt)
    omp = 1.0 - pt
    focal = -(omp * omp) * logpt
    incl = lab_row >= 0
    sf = jnp.sum(jnp.where(incl, focal, 0.0), axis=1, keepdims=True)
    ninc = jnp.sum(incl.astype(jnp.float32), axis=1, keepdims=True)

    acc_ref[0:1, 0:1] = acc_ref[0:1, 0:1] + sd
    acc_ref[0:1, 1:2] = acc_ref[0:1, 1:2] + npos
    acc_ref[0:1, 2:3] = acc_ref[0:1, 2:3] + sf
    acc_ref[0:1, 3:4] = acc_ref[0:1, 3:4] + ninc

    @pl.when(jnp.logical_and(b == BATCH - 1, s == NBLK - 1))
    def _fin():
        np_ = jnp.maximum(acc_ref[0:1, 1:2], 1.0)
        conf = (acc_ref[0:1, 2:3] / jnp.maximum(acc_ref[0:1, 3:4], 1.0)) / np_
        out_ref[...] = conf + ALPHA * (acc_ref[0:1, 0:1] / np_)


def kernel(predicted_locs, predicted_scores, boxes, labels, priors_cxcy):
    priors_t = jnp.transpose(priors_cxcy, (1, 0))
    padsc = jnp.concatenate(
        [jnp.full((2, PADSC - N_PRIORS), 2.0, jnp.float32),
         jnp.zeros((2, PADSC - N_PRIORS), jnp.float32)], axis=0)
    priors_sc = jnp.concatenate([priors_t, padsc], axis=1)
    boxes_sc = jnp.transpose(boxes, (0, 2, 1))

    lab_r, tl = _sc_assign(boxes_sc, labels.reshape(BATCH, 1, N_OBJ), priors_sc)

    locs_t = jnp.transpose(predicted_locs, (0, 2, 1))
    pad = jnp.concatenate(
        [jnp.full((2, PADP - N_PRIORS), 2.0, jnp.float32),
         jnp.zeros((2, PADP - N_PRIORS), jnp.float32)], axis=0)
    priors_tp = jnp.concatenate([priors_t, pad], axis=1)

    out = pl.pallas_call(
        _mbox_kernel,
        grid=(BATCH, NBLK),
        in_specs=[
            pl.BlockSpec((1, 4, BLK), lambda b, s: (b, 0, s)),
            pl.BlockSpec((1, BLK, N_CLASSES), lambda b, s: (b, s, 0)),
            pl.BlockSpec((1, 1, PADP), lambda b, s: (b, 0, 0)),
            pl.BlockSpec((1, 4, PADP), lambda b, s: (b, 0, 0)),
            pl.BlockSpec((4, PADP), lambda b, s: (0, 0)),
        ],
        out_specs=pl.BlockSpec((1, 1), lambda b, s: (0, 0)),
        out_shape=jax.ShapeDtypeStruct((1, 1), jnp.float32),
        scratch_shapes=[
            pltpu.VMEM((1, 128), jnp.float32),
        ],
        compiler_params=pltpu.CompilerParams(
            dimension_semantics=("arbitrary", "arbitrary")),
    )(locs_t, predicted_scores, lab_r, tl, priors_tp)
    return out[0, 0]


# speedup vs baseline: 1.1767x; 1.1767x over previous
import functools

import jax
import jax.numpy as jnp
from jax import lax
from jax.experimental import pallas as pl
from jax.experimental.pallas import tpu as pltpu
from jax.experimental.pallas import tpu_sc as plsc

BATCH = 16
N_PRIORS = 8732
N_CLASSES = 81
N_OBJ = 16
THRESHOLD = 0.5
ALPHA = 25.0
EPS = 1e-7

BLK = 2048
NBLK = 5
PADP = BLK * NBLK
PADSC = 8736
NCHUNK = PADSC // 16
NPADCHUNK = (PADP - PADSC) // 16


def _sc_assign_body(boxes_hbm, labels_hbm, priors_hbm, lab_out, tl_out,
                    pri_v, box_v, labl_v, lab_v, tl_v):
    wid = lax.axis_index("s") * 2 + lax.axis_index("c")

    @pl.when(wid < BATCH)
    def _():
        b = wid
        pltpu.sync_copy(boxes_hbm.at[b], box_v)
        pltpu.sync_copy(labels_hbm.at[b], labl_v)
        pltpu.sync_copy(priors_hbm, pri_v)
        lanes = lax.iota(jnp.int32, 16)
        zzf = jnp.zeros((16,), jnp.float32)

        def splat(k, i):
            return zzf + jnp.sum(jnp.where(lanes == i, box_v[k, :], 0.0))

        bx0 = [splat(0, i) for i in range(N_OBJ)]
        by0 = [splat(1, i) for i in range(N_OBJ)]
        bx1 = [splat(2, i) for i in range(N_OBJ)]
        by1 = [splat(3, i) for i in range(N_OBJ)]
        barea = [(bx1[i] - bx0[i]) * (by1[i] - by0[i]) for i in range(N_OBJ)]

        def body(j, acc):
            base = j * 16
            pcx = pri_v[0, pl.ds(base, 16)]
            pcy = pri_v[1, pl.ds(base, 16)]
            pw = pri_v[2, pl.ds(base, 16)]
            ph = pri_v[3, pl.ds(base, 16)]
            px0 = pcx - pw * 0.5
            py0 = pcy - ph * 0.5
            px1 = pcx + pw * 0.5
            py1 = pcy + ph * 0.5
            parea = (px1 - px0) * (py1 - py0)
            idxv = base + lanes
            nf = jnp.full((16,), -1.0, jnp.float32)
            df = jnp.ones((16,), jnp.float32)
            of = jnp.zeros((16,), jnp.int32)
            for i in range(N_OBJ):
                iw = jnp.maximum(jnp.minimum(bx1[i], px1)
                                 - jnp.maximum(bx0[i], px0), 0.0)
                ih = jnp.maximum(jnp.minimum(by1[i], py1)
                                 - jnp.maximum(by0[i], py0), 0.0)
                n = iw * ih
                dd = barea[i] + parea - n + EPS
                better = n * df > nf * dd
                of = jnp.where(better, i, of)
                nf = jnp.where(better, n, nf)
                df = jnp.where(better, dd, df)
            labv = plsc.load_gather(labl_v, [jnp.zeros((16,), jnp.int32), of])
            labv = jnp.where(nf < THRESHOLD * df, -1, labv)
            labv = jnp.where(nf < (THRESHOLD - 0.1) * df, 0, labv)
            labv = jnp.where(idxv < N_PRIORS, labv, -1)
            lab_v[0, pl.ds(base, 16)] = labv
            for k in range(4):
                tl_v[k, pl.ds(base, 16)] = plsc.load_gather(
                    box_v, [jnp.full((16,), k, jnp.int32), of])
            return acc

        lax.fori_loop(0, NCHUNK, body, 0)

        rn, rd, ri = [], [], []
        for i in range(N_OBJ):
            def rbody(j, carry, i=i):
                crn, crd, cri = carry
                base = j * 16
                pcx = pri_v[0, pl.ds(base, 16)]
                pcy = pri_v[1, pl.ds(base, 16)]
                pw = pri_v[2, pl.ds(base, 16)]
                ph = pri_v[3, pl.ds(base, 16)]
                px0 = pcx - pw * 0.5
                py0 = pcy - ph * 0.5
                px1 = pcx + pw * 0.5
                py1 = pcy + ph * 0.5
                parea = (px1 - px0) * (py1 - py0)
                iw = jnp.maximum(jnp.minimum(bx1[i], px1)
                                 - jnp.maximum(bx0[i], px0), 0.0)
                ih = jnp.maximum(jnp.minimum(by1[i], py1)
                                 - jnp.maximum(by0[i], py0), 0.0)
                n = iw * ih
                dd = barea[i] + parea - n + EPS
                upd = n * crd > crn * dd
                return (jnp.where(upd, n, crn),
                        jnp.where(upd, dd, crd),
                        jnp.where(upd, base + lanes, cri))

            crn, crd, cri = lax.fori_loop(
                0, NCHUNK, rbody,
                (jnp.full((16,), -1.0, jnp.float32),
                 jnp.ones((16,), jnp.float32),
                 jnp.zeros((16,), jnp.int32)))
            rn.append(crn)
            rd.append(crd)
            ri.append(cri)

        def padbody(j, acc):
            base = PADSC + j * 16
            lab_v[0, pl.ds(base, 16)] = jnp.full((16,), -1, jnp.int32)
            for k in range(4):
                tl_v[k, pl.ds(base, 16)] = jnp.zeros((16,), jnp.float32)
            return acc

        lax.fori_loop(0, NPADCHUNK, padbody, 0)

        mask0 = lanes == 0
        z16 = jnp.zeros((16,), jnp.int32)
        for i in range(N_OBJ):
            bn = jnp.full((16,), -1.0, jnp.float32)
            bd = jnp.ones((16,), jnp.float32)
            bi = jnp.full((16,), PADP, jnp.int32)
            for l in range(16):
                lsel = lanes == l
                nl = zzf + jnp.sum(jnp.where(lsel, rn[i], 0.0))
                dl = zzf + jnp.sum(jnp.where(lsel, rd[i], 0.0))
                il = z16 + jnp.sum(jnp.where(lsel, ri[i], 0))
                cl = nl * bd
                cr = bn * dl
                better = jnp.logical_or(cl > cr,
                                        jnp.logical_and(cl == cr, il < bi))
                bn = jnp.where(better, nl, bn)
                bd = jnp.where(better, dl, bd)
                bi = jnp.where(better, il, bi)
            pfi = jnp.min(bi)
            pfv = jnp.full((16,), 0, jnp.int32) + pfi
            li = z16 + jnp.sum(jnp.where(lanes == i, labl_v[0, :], 0))
            plsc.store_scatter(lab_v, [jnp.zeros((16,), jnp.int32), pfv], li, mask=mask0)
            for k in range(4):
                plsc.store_scatter(tl_v, [jnp.full((16,), k, jnp.int32), pfv],
                                   splat(k, i), mask=mask0)

        pltpu.sync_copy(lab_v, lab_out.at[b])
        pltpu.sync_copy(tl_v, tl_out.at[b])


def _sc_assign(boxes_sc, labels, priors_sc):
    fn = pl.kernel(
        _sc_assign_body,
        mesh=plsc.VectorSubcoreMesh(core_axis_name="c", subcore_axis_name="s"),
        out_type=[jax.ShapeDtypeStruct((BATCH, 1, PADP), jnp.int32),
                  jax.ShapeDtypeStruct((BATCH, 4, PADP), jnp.float32)],
        scratch_types=[
            pltpu.VMEM((4, PADSC), jnp.float32),
            pltpu.VMEM((4, N_OBJ), jnp.float32),
            pltpu.VMEM((1, N_OBJ), jnp.int32),
            pltpu.VMEM((1, PADP), jnp.int32),
            pltpu.VMEM((4, PADP), jnp.float32),
        ],
        compiler_params=pltpu.CompilerParams(needs_layout_passes=False),
    )
    return fn(boxes_sc, labels, priors_sc)


def _mbox_kernel(locs_ref, scores_ref, lab_in, tl_in, priors_ref,
                 out_ref, acc_ref):
    b = pl.program_id(0)
    s = pl.program_id(1)

    @pl.when(jnp.logical_and(b == 0, s == 0))
    def _init():
        acc_ref[...] = jnp.zeros_like(acc_ref)

    g = locs_ref[0, :, :]
    pr = priors_ref[:, pl.ds(s * BLK, BLK)]
    pcx, pcy, pw, ph = pr[0:1], pr[1:2], pr[2:3], pr[3:4]
    cx = g[0:1] * pw / 10.0 + pcx
    cy = g[1:2] * ph / 10.0 + pcy
    w = jnp.exp(g[2:3] / 5.0) * pw
    h = jnp.exp(g[3:4] / 5.0) * ph
    dx0 = cx - w / 2.0
    dy0 = cy - h / 2.0
    dx1 = cx + w / 2.0
    dy1 = cy + h / 2.0

    t = tl_in[0, :, pl.ds(s * BLK, BLK)]
    tx0, ty0, tx1, ty1 = t[0:1], t[1:2], t[2:3], t[3:4]
    inter = (jnp.clip(jnp.minimum(dx1, tx1) - jnp.maximum(dx0, tx0), 0.0, None)
             * jnp.clip(jnp.minimum(dy1, ty1) - jnp.maximum(dy0, ty0), 0.0, None))
    ap = (dx1 - dx0) * (dy1 - dy0)
    at_ = (tx1 - tx0) * (ty1 - ty0)
    iou = inter / (ap + at_ - inter + EPS)
    rho2 = (((dx0 + dx1) - (tx0 + tx1)) / 2.0) ** 2 + (((dy0 + dy1) - (ty0 + ty1)) / 2.0) ** 2
    ex = jnp.maximum(dx1, tx1) - jnp.minimum(dx0, tx0)
    ey = jnp.maximum(dy1, ty1) - jnp.minimum(dy0, ty0)
    c2 = ex * ex + ey * ey + EPS
    diou = 1.0 - (iou - rho2 / c2)

    lab_row = lab_in[0, 0:1, pl.ds(s * BLK, BLK)]
    posr = lab_row > 0
    sd = jnp.sum(jnp.where(posr, diou, 0.0), axis=1, keepdims=True)
    npos = jnp.sum(posr.astype(jnp.float32), axis=1, keepdims=True)

    St = jnp.transpose(scores_ref[0, :, :], (1, 0))
    tgt = jnp.clip(lab_row, 0, N_CLASSES - 1)
    cid = jax.lax.broadcasted_iota(jnp.int32, (N_CLASSES, BLK), 0)
    ones_c = jnp.ones((1, N_CLASSES), jnp.float32)
    se = jax.lax.dot_general(ones_c, jnp.exp(St), (((1,), (0,)), ((), ())),
                             preferred_element_type=jnp.float32)
    s_tgt = jax.lax.dot_general(ones_c, jnp.where(cid == tgt, St, 0.0),
                                (((1,), (0,)), ((), ())),
                                preferred_element_type=jnp.float32)
    logpt = s_tgt - jnp.log(se)
    pt = jnp.exp(logpt)
    omp = 1.0 - pt
    focal = -(omp * omp) * logpt
    incl = lab_row >= 0
    sf = jnp.sum(jnp.where(incl, focal, 0.0), axis=1, keepdims=True)
    ninc = jnp.sum(incl.astype(jnp.float32), axis=1, keepdims=True)

    acc_ref[0:1, 0:1] = acc_ref[0:1, 0:1] + sd
    acc_ref[0:1, 1:2] = acc_ref[0:1, 1:2] + npos
    acc_ref[0:1, 2:3] = acc_ref[0:1, 2:3] + sf
    acc_ref[0:1, 3:4] = acc_ref[0:1, 3:4] + ninc

    @pl.when(jnp.logical_and(b == BATCH - 1, s == NBLK - 1))
    def _fin():
        np_ = jnp.maximum(acc_ref[0:1, 1:2], 1.0)
        conf = (acc_ref[0:1, 2:3] / jnp.maximum(acc_ref[0:1, 3:4], 1.0)) / np_
        out_ref[...] = conf + ALPHA * (acc_ref[0:1, 0:1] / np_)


def kernel(predicted_locs, predicted_scores, boxes, labels, priors_cxcy):
    priors_t = jnp.transpose(priors_cxcy, (1, 0))
    padsc = jnp.concatenate(
        [jnp.full((2, PADSC - N_PRIORS), 2.0, jnp.float32),
         jnp.zeros((2, PADSC - N_PRIORS), jnp.float32)], axis=0)
    priors_sc = jnp.concatenate([priors_t, padsc], axis=1)
    boxes_sc = jnp.transpose(boxes, (0, 2, 1))

    lab_r, tl = _sc_assign(boxes_sc, labels.reshape(BATCH, 1, N_OBJ), priors_sc)

    locs_t = jnp.transpose(predicted_locs, (0, 2, 1))
    pad = jnp.concatenate(
        [jnp.full((2, PADP - N_PRIORS), 2.0, jnp.float32),
         jnp.zeros((2, PADP - N_PRIORS), jnp.float32)], axis=0)
    priors_tp = jnp.concatenate([priors_t, pad], axis=1)

    out = pl.pallas_call(
        _mbox_kernel,
        grid=(BATCH, NBLK),
        in_specs=[
            pl.BlockSpec((1, 4, BLK), lambda b, s: (b, 0, s)),
            pl.BlockSpec((1, BLK, N_CLASSES), lambda b, s: (b, s, 0)),
            pl.BlockSpec((1, 1, PADP), lambda b, s: (b, 0, 0)),
            pl.BlockSpec((1, 4, PADP), lambda b, s: (b, 0, 0)),
            pl.BlockSpec((4, PADP), lambda b, s: (0, 0)),
        ],
        out_specs=pl.BlockSpec((1, 1), lambda b, s: (0, 0)),
        out_shape=jax.ShapeDtypeStruct((1, 1), jnp.float32),
        scratch_shapes=[
            pltpu.VMEM((1, 128), jnp.float32),
        ],
        compiler_params=pltpu.CompilerParams(
            dimension_semantics=("arbitrary", "arbitrary")),
    )(locs_t, predicted_scores, lab_r, tl, priors_tp)
    return out[0, 0]
